# baseline (speedup 1.0000x reference)
import functools

import jax
import jax.numpy as jnp
from jax import lax
from jax.experimental import pallas as pl
from jax.experimental.pallas import tpu as pltpu
from jax.experimental.pallas import tpu_sc as plsc

C, H, W = 192, 384, 384
L = 16
NCHUNK = W // L
RBLK = 64
NBLK = H // RBLK
NWORK = 32


def _sc_body(cs, x_hbm, out_hbm, in0, in1, out0, out1, si0, si1, so0, so1):
    cpw = cs // NWORK
    nt = cpw * NBLK
    wid = lax.axis_index("s") * 2 + lax.axis_index("c")
    iota = lax.iota(jnp.int32, L)
    neg = jnp.full((L,), -jnp.inf, jnp.float32)
    idx15 = jnp.full((L,), L - 1, jnp.int32)
    rev_idx = [(W - 1 - c * L) - iota for c in range(NCHUNK)]

    def task_slice(t):
        ch = wid * cpw + t // NBLK
        row0 = ((NBLK - 1) - (t % NBLK)) * RBLK
        return ch, row0

    def start_in(t, buf, sem):
        ch, row0 = task_slice(t)
        pltpu.async_copy(x_hbm.at[ch, pl.ds(row0, RBLK), :], buf, sem)

    def wait_in(buf, sem):
        pltpu.make_async_copy(x_hbm.at[0, pl.ds(0, RBLK), :], buf, sem).wait()

    def start_out(t, buf, sem):
        ch, row0 = task_slice(t)
        pltpu.async_copy(buf, out_hbm.at[ch, pl.ds(row0, RBLK), :], sem)

    def wait_out(buf, sem):
        pltpu.make_async_copy(buf, out_hbm.at[0, pl.ds(0, RBLK), :], sem).wait()

    def compute(t, in_v, out_v, colmax):
        fresh = (t % NBLK) == 0
        colmax = tuple(jnp.where(fresh, neg, cm) for cm in colmax)

        def row_body(i, colmax):
            r = RBLK - 1 - i
            rfull = jnp.full((L,), 0, jnp.int32) + r
            new_cm = []
            carry = neg
            for c in range(NCHUNK):
                y = plsc.load_gather(in_v, [rfull, rev_idx[c]])
                p = plsc.cummax(y)
                left = jnp.maximum(p, carry)
                carry = lax.gather(
                    left,
                    idx15[:, None],
                    lax.GatherDimensionNumbers(
                        offset_dims=(),
                        collapsed_slice_dims=(0,),
                        start_index_map=(0,),
                    ),
                    slice_sizes=(1,),
                    mode=lax.GatherScatterMode.PROMISE_IN_BOUNDS,
                )
                cm = jnp.maximum(colmax[c], y)
                new_cm.append(cm)
                plsc.store_scatter(out_v, [rfull, rev_idx[c]], left + cm)
            return tuple(new_cm)

        return plsc.parallel_loop(0, RBLK, unroll=2, carry=colmax)(row_body)

    start_in(0, in0, si0)

    def pair_body(s, colmax):
        t0 = 2 * s
        t1 = t0 + 1
        start_in(t1, in1, si1)
        wait_in(in0, si0)

        @pl.when(s > 0)
        def _():
            wait_out(out0, so0)

        colmax = compute(t0, in0, out0, colmax)
        start_out(t0, out0, so0)

        @pl.when(t1 + 1 < nt)
        def _():
            start_in(t1 + 1, in0, si0)

        wait_in(in1, si1)

        @pl.when(s > 0)
        def _():
            wait_out(out1, so1)

        colmax = compute(t1, in1, out1, colmax)
        start_out(t1, out1, so1)
        return colmax

    lax.fori_loop(0, nt // 2, pair_body, (neg,) * NCHUNK)
    wait_out(out0, so0)
    wait_out(out1, so1)


@functools.cache
def _make_sc_kernel(cs):
    @functools.partial(
        pl.kernel,
        out_type=jax.ShapeDtypeStruct((cs, H, W), jnp.float32),
        mesh=plsc.VectorSubcoreMesh(core_axis_name="c", subcore_axis_name="s"),
        scratch_types=[
            pltpu.VMEM((RBLK, W), jnp.float32),
            pltpu.VMEM((RBLK, W), jnp.float32),
            pltpu.VMEM((RBLK, W), jnp.float32),
            pltpu.VMEM((RBLK, W), jnp.float32),
            pltpu.SemaphoreType.DMA,
            pltpu.SemaphoreType.DMA,
            pltpu.SemaphoreType.DMA,
            pltpu.SemaphoreType.DMA,
        ],
        compiler_params=pltpu.CompilerParams(
            use_tc_tiling_on_sc=False, needs_layout_passes=False
        ),
    )
    def sc_kernel(x_hbm, out_hbm, in0, in1, out0, out1, si0, si1, so0, so1):
        _sc_body(cs, x_hbm, out_hbm, in0, in1, out0, out1, si0, si1, so0, so1)

    return sc_kernel


TCB = 8


def _tc_block(x_ref, o_ref):
    v = x_ref[...]
    t = v
    off = 1
    while off < H:
        pad = jnp.full((TCB, off, W), -jnp.inf, jnp.float32)
        t = jnp.maximum(t, jnp.concatenate([t[:, off:, :], pad], axis=1))
        off *= 2
    left = v
    off = 1
    while off < W:
        pad = jnp.full((TCB, H, off), -jnp.inf, jnp.float32)
        left = jnp.maximum(left, jnp.concatenate([left[:, :, off:], pad], axis=2))
        off *= 2
    o_ref[...] = t + left


def _tc_kernel(x):
    ct = x.shape[0]
    return pl.pallas_call(
        _tc_block,
        out_shape=jax.ShapeDtypeStruct(x.shape, x.dtype),
        grid=(ct // TCB,),
        in_specs=[pl.BlockSpec((TCB, H, W), lambda i: (i, 0, 0))],
        out_specs=pl.BlockSpec((TCB, H, W), lambda i: (i, 0, 0)),
    )(x)


CS = 64


def kernel(x):
    if CS == 0:
        return _tc_kernel(x)
    if CS == C:
        return _make_sc_kernel(C)(x)
    sc = _make_sc_kernel(CS)(x[:CS])
    tc = _tc_kernel(x[CS:])
    return jnp.concatenate([sc, tc], axis=0)


# speedup vs baseline: 1.6390x; 1.6390x over previous
import functools

import jax
import jax.numpy as jnp
from jax import lax
from jax.experimental import pallas as pl
from jax.experimental.pallas import tpu as pltpu
from jax.experimental.pallas import tpu_sc as plsc

C, H, W = 192, 384, 384
L = 16
NCHUNK = W // L
RBLK = 64
NBLK = H // RBLK
NWORK = 32


def _sc_body(cs, x_hbm, out_hbm, in0, in1, out0, out1, si0, si1, so0, so1):
    cpw = cs // NWORK
    nt = cpw * NBLK
    wid = lax.axis_index("s") * 2 + lax.axis_index("c")
    neg = jnp.full((L,), -jnp.inf, jnp.float32)
    idx15 = jnp.full((L,), L - 1, jnp.int32)

    def task_slice(t):
        ch = wid * cpw + t // NBLK
        row0 = ((NBLK - 1) - (t % NBLK)) * RBLK
        return ch, row0

    def start_in(t, buf, sem):
        ch, row0 = task_slice(t)
        pltpu.async_copy(x_hbm.at[ch, pl.ds(row0, RBLK), :], buf, sem)

    def wait_in(buf, sem):
        pltpu.make_async_copy(x_hbm.at[0, pl.ds(0, RBLK), :], buf, sem).wait()

    def start_out(t, buf, sem):
        ch, row0 = task_slice(t)
        pltpu.async_copy(buf, out_hbm.at[ch, pl.ds(row0, RBLK), :], sem)

    def wait_out(buf, sem):
        pltpu.make_async_copy(buf, out_hbm.at[0, pl.ds(0, RBLK), :], sem).wait()

    def splat15(v):
        return lax.gather(
            v,
            idx15[:, None],
            lax.GatherDimensionNumbers(
                offset_dims=(),
                collapsed_slice_dims=(0,),
                start_index_map=(0,),
            ),
            slice_sizes=(1,),
            mode=lax.GatherScatterMode.PROMISE_IN_BOUNDS,
        )

    def compute(t, in_v, out_v, colmax):
        fresh = (t % NBLK) == 0
        colmax = tuple(jnp.where(fresh, neg, cm) for cm in colmax)

        def row_body(i, colmax):
            r = RBLK - 1 - i
            new_cm = []
            carry = neg
            for c in range(NCHUNK - 1, -1, -1):
                y = in_v[r, pl.ds(c * L, L)]
                p = plsc.cummax(jnp.flip(y, 0))
                left = jnp.maximum(jnp.flip(p, 0), carry)
                carry = jnp.maximum(carry, splat15(p))
                cm = jnp.maximum(colmax[c], y)
                new_cm.append(cm)
                out_v[r, pl.ds(c * L, L)] = left + cm
            new_cm.reverse()
            return tuple(new_cm)

        return plsc.parallel_loop(0, RBLK, unroll=2, carry=colmax)(row_body)

    start_in(0, in0, si0)

    def pair_body(s, colmax):
        t0 = 2 * s
        t1 = t0 + 1
        start_in(t1, in1, si1)
        wait_in(in0, si0)

        @pl.when(s > 0)
        def _():
            wait_out(out0, so0)

        colmax = compute(t0, in0, out0, colmax)
        start_out(t0, out0, so0)

        @pl.when(t1 + 1 < nt)
        def _():
            start_in(t1 + 1, in0, si0)

        wait_in(in1, si1)

        @pl.when(s > 0)
        def _():
            wait_out(out1, so1)

        colmax = compute(t1, in1, out1, colmax)
        start_out(t1, out1, so1)
        return colmax

    lax.fori_loop(0, nt // 2, pair_body, (neg,) * NCHUNK)
    wait_out(out0, so0)
    wait_out(out1, so1)


@functools.cache
def _make_sc_kernel(cs):
    @functools.partial(
        pl.kernel,
        out_type=jax.ShapeDtypeStruct((cs, H, W), jnp.float32),
        mesh=plsc.VectorSubcoreMesh(core_axis_name="c", subcore_axis_name="s"),
        scratch_types=[
            pltpu.VMEM((RBLK, W), jnp.float32),
            pltpu.VMEM((RBLK, W), jnp.float32),
            pltpu.VMEM((RBLK, W), jnp.float32),
            pltpu.VMEM((RBLK, W), jnp.float32),
            pltpu.SemaphoreType.DMA,
            pltpu.SemaphoreType.DMA,
            pltpu.SemaphoreType.DMA,
            pltpu.SemaphoreType.DMA,
        ],
        compiler_params=pltpu.CompilerParams(
            use_tc_tiling_on_sc=True, needs_layout_passes=False
        ),
    )
    def sc_kernel(x_hbm, out_hbm, in0, in1, out0, out1, si0, si1, so0, so1):
        _sc_body(cs, x_hbm, out_hbm, in0, in1, out0, out1, si0, si1, so0, so1)

    return sc_kernel


TCB = 8


def _tc_block(x_ref, o_ref):
    v = x_ref[...]
    t = v
    off = 1
    while off < H:
        pad = jnp.full((TCB, off, W), -jnp.inf, jnp.float32)
        t = jnp.maximum(t, jnp.concatenate([t[:, off:, :], pad], axis=1))
        off *= 2
    left = v
    off = 1
    while off < W:
        pad = jnp.full((TCB, H, off), -jnp.inf, jnp.float32)
        left = jnp.maximum(left, jnp.concatenate([left[:, :, off:], pad], axis=2))
        off *= 2
    o_ref[...] = t + left


def _tc_kernel(x, c0):
    ct = x.shape[0] - c0
    return pl.pallas_call(
        _tc_block,
        out_shape=jax.ShapeDtypeStruct(x.shape, x.dtype),
        grid=(ct // TCB,),
        in_specs=[pl.BlockSpec((TCB, H, W), lambda i: (i + c0 // TCB, 0, 0))],
        out_specs=pl.BlockSpec((TCB, H, W), lambda i: (i + c0 // TCB, 0, 0)),
    )(x)


CS = 96


def kernel(x):
    if CS == 0:
        return _tc_kernel(x, 0)
    sc = _make_sc_kernel(CS)(x)
    if CS == C:
        return sc
    tc = _tc_kernel(x, CS)
    return lax.dynamic_update_slice(tc, sc, (0, 0, 0))


# speedup vs baseline: 2.0306x; 1.2389x over previous
import functools

import jax
import jax.numpy as jnp
from jax import lax
from jax.experimental import pallas as pl
from jax.experimental.pallas import tpu as pltpu
from jax.experimental.pallas import tpu_sc as plsc

C, H, W = 192, 384, 384
L = 16
NCHUNK = W // L
RBLK = 64
NBLK = H // RBLK
NWORK = 32


def _sc_body(cs, x_hbm, out_hbm, in0, in1, out0, out1, si0, si1, so0, so1):
    cpw = cs // NWORK
    nt = cpw * NBLK
    wid = lax.axis_index("s") * 2 + lax.axis_index("c")
    neg = jnp.full((L,), -jnp.inf, jnp.float32)
    idx15 = jnp.full((L,), L - 1, jnp.int32)

    def task_slice(t):
        ch = wid * cpw + t // NBLK
        row0 = ((NBLK - 1) - (t % NBLK)) * RBLK
        return ch, row0

    def start_in(t, buf, sem):
        ch, row0 = task_slice(t)
        pltpu.async_copy(x_hbm.at[ch, pl.ds(row0, RBLK), :], buf, sem)

    def wait_in(buf, sem):
        pltpu.make_async_copy(x_hbm.at[0, pl.ds(0, RBLK), :], buf, sem).wait()

    def start_out(t, buf, sem):
        ch, row0 = task_slice(t)
        pltpu.async_copy(buf, out_hbm.at[ch, pl.ds(row0, RBLK), :], sem)

    def wait_out(buf, sem):
        pltpu.make_async_copy(buf, out_hbm.at[0, pl.ds(0, RBLK), :], sem).wait()

    def splat15(v):
        return lax.gather(
            v,
            idx15[:, None],
            lax.GatherDimensionNumbers(
                offset_dims=(),
                collapsed_slice_dims=(0,),
                start_index_map=(0,),
            ),
            slice_sizes=(1,),
            mode=lax.GatherScatterMode.PROMISE_IN_BOUNDS,
        )

    def compute(t, in_v, out_v, colmax):
        fresh = (t % NBLK) == 0
        colmax = tuple(jnp.where(fresh, neg, cm) for cm in colmax)

        def row_body(i, colmax):
            r = RBLK - 1 - i
            new_cm = []
            carry = neg
            for c in range(NCHUNK - 1, -1, -1):
                y = in_v[r, pl.ds(c * L, L)]
                p = plsc.cummax(jnp.flip(y, 0))
                left = jnp.maximum(jnp.flip(p, 0), carry)
                carry = jnp.maximum(carry, splat15(p))
                cm = jnp.maximum(colmax[c], y)
                new_cm.append(cm)
                out_v[r, pl.ds(c * L, L)] = left + cm
            new_cm.reverse()
            return tuple(new_cm)

        return plsc.parallel_loop(0, RBLK, unroll=2, carry=colmax)(row_body)

    start_in(0, in0, si0)

    def pair_body(s, colmax):
        t0 = 2 * s
        t1 = t0 + 1
        start_in(t1, in1, si1)
        wait_in(in0, si0)

        @pl.when(s > 0)
        def _():
            wait_out(out0, so0)

        colmax = compute(t0, in0, out0, colmax)
        start_out(t0, out0, so0)

        @pl.when(t1 + 1 < nt)
        def _():
            start_in(t1 + 1, in0, si0)

        wait_in(in1, si1)

        @pl.when(s > 0)
        def _():
            wait_out(out1, so1)

        colmax = compute(t1, in1, out1, colmax)
        start_out(t1, out1, so1)
        return colmax

    lax.fori_loop(0, nt // 2, pair_body, (neg,) * NCHUNK)
    wait_out(out0, so0)
    wait_out(out1, so1)


@functools.cache
def _make_sc_kernel(cs):
    @functools.partial(
        pl.kernel,
        out_type=jax.ShapeDtypeStruct((cs, H, W), jnp.float32),
        mesh=plsc.VectorSubcoreMesh(core_axis_name="c", subcore_axis_name="s"),
        scratch_types=[
            pltpu.VMEM((RBLK, W), jnp.float32),
            pltpu.VMEM((RBLK, W), jnp.float32),
            pltpu.VMEM((RBLK, W), jnp.float32),
            pltpu.VMEM((RBLK, W), jnp.float32),
            pltpu.SemaphoreType.DMA,
            pltpu.SemaphoreType.DMA,
            pltpu.SemaphoreType.DMA,
            pltpu.SemaphoreType.DMA,
        ],
        compiler_params=pltpu.CompilerParams(
            use_tc_tiling_on_sc=True, needs_layout_passes=False
        ),
    )
    def sc_kernel(x_hbm, out_hbm, in0, in1, out0, out1, si0, si1, so0, so1):
        _sc_body(cs, x_hbm, out_hbm, in0, in1, out0, out1, si0, si1, so0, so1)

    return sc_kernel


TCB = 8


def _tc_block(x_ref, o_ref):
    v = x_ref[...]
    t = v
    off = 1
    while off < H:
        pad = jnp.full((TCB, off, W), -jnp.inf, jnp.float32)
        t = jnp.maximum(t, jnp.concatenate([t[:, off:, :], pad], axis=1))
        off *= 2
    left = v
    off = 1
    while off < W:
        pad = jnp.full((TCB, H, off), -jnp.inf, jnp.float32)
        left = jnp.maximum(left, jnp.concatenate([left[:, :, off:], pad], axis=2))
        off *= 2
    o_ref[...] = t + left


def _tc_kernel(x, c0):
    ct = x.shape[0] - c0
    return pl.pallas_call(
        _tc_block,
        out_shape=jax.ShapeDtypeStruct(x.shape, x.dtype),
        grid=(ct // TCB,),
        in_specs=[pl.BlockSpec((TCB, H, W), lambda i: (i + c0 // TCB, 0, 0))],
        out_specs=pl.BlockSpec((TCB, H, W), lambda i: (i + c0 // TCB, 0, 0)),
    )(x)


CS = 64


def kernel(x):
    if CS == 0:
        return _tc_kernel(x, 0)
    sc = _make_sc_kernel(CS)(x)
    if CS == C:
        return sc
    tc = _tc_kernel(x, CS)
    return lax.dynamic_update_slice(tc, sc, (0, 0, 0))


# speedup vs baseline: 2.1589x; 1.0632x over previous
import functools

import jax
import jax.numpy as jnp
from jax import lax
from jax.experimental import pallas as pl
from jax.experimental.pallas import tpu as pltpu
from jax.experimental.pallas import tpu_sc as plsc

C, H, W = 192, 384, 384
L = 16
NCHUNK = W // L
RBLK = 64
NBLK = H // RBLK
NWORK = 32


def _sc_body(cs, x_hbm, out_hbm, in0, in1, out0, out1, si0, si1, so0, so1):
    cpw = cs // NWORK
    nt = cpw * NBLK
    wid = lax.axis_index("s") * 2 + lax.axis_index("c")
    neg = jnp.full((L,), -jnp.inf, jnp.float32)
    idx15 = jnp.full((L,), L - 1, jnp.int32)

    def task_slice(t):
        ch = wid * cpw + t // NBLK
        row0 = ((NBLK - 1) - (t % NBLK)) * RBLK
        return ch, row0

    def start_in(t, buf, sem):
        ch, row0 = task_slice(t)
        pltpu.async_copy(x_hbm.at[ch, pl.ds(row0, RBLK), :], buf, sem)

    def wait_in(buf, sem):
        pltpu.make_async_copy(x_hbm.at[0, pl.ds(0, RBLK), :], buf, sem).wait()

    def start_out(t, buf, sem):
        ch, row0 = task_slice(t)
        pltpu.async_copy(buf, out_hbm.at[ch, pl.ds(row0, RBLK), :], sem)

    def wait_out(buf, sem):
        pltpu.make_async_copy(buf, out_hbm.at[0, pl.ds(0, RBLK), :], sem).wait()

    def splat15(v):
        return lax.gather(
            v,
            idx15[:, None],
            lax.GatherDimensionNumbers(
                offset_dims=(),
                collapsed_slice_dims=(0,),
                start_index_map=(0,),
            ),
            slice_sizes=(1,),
            mode=lax.GatherScatterMode.PROMISE_IN_BOUNDS,
        )

    def compute(t, in_v, out_v, colmax):
        fresh = (t % NBLK) == 0
        colmax = tuple(jnp.where(fresh, neg, cm) for cm in colmax)

        def row_body(i, colmax):
            r = RBLK - 1 - i
            new_cm = []
            carry = neg
            for c in range(NCHUNK - 1, -1, -1):
                y = in_v[r, pl.ds(c * L, L)]
                p = plsc.cummax(jnp.flip(y, 0))
                left = jnp.maximum(jnp.flip(p, 0), carry)
                carry = jnp.maximum(carry, splat15(p))
                cm = jnp.maximum(colmax[c], y)
                new_cm.append(cm)
                out_v[r, pl.ds(c * L, L)] = left + cm
            new_cm.reverse()
            return tuple(new_cm)

        return plsc.parallel_loop(0, RBLK, unroll=2, carry=colmax)(row_body)

    start_in(0, in0, si0)

    def pair_body(s, colmax):
        t0 = 2 * s
        t1 = t0 + 1
        start_in(t1, in1, si1)
        wait_in(in0, si0)

        @pl.when(s > 0)
        def _():
            wait_out(out0, so0)

        colmax = compute(t0, in0, out0, colmax)
        start_out(t0, out0, so0)

        @pl.when(t1 + 1 < nt)
        def _():
            start_in(t1 + 1, in0, si0)

        wait_in(in1, si1)

        @pl.when(s > 0)
        def _():
            wait_out(out1, so1)

        colmax = compute(t1, in1, out1, colmax)
        start_out(t1, out1, so1)
        return colmax

    lax.fori_loop(0, nt // 2, pair_body, (neg,) * NCHUNK)
    wait_out(out0, so0)
    wait_out(out1, so1)


@functools.cache
def _make_sc_kernel(cs):
    @functools.partial(
        pl.kernel,
        out_type=jax.ShapeDtypeStruct((cs, H, W), jnp.float32),
        mesh=plsc.VectorSubcoreMesh(core_axis_name="c", subcore_axis_name="s"),
        scratch_types=[
            pltpu.VMEM((RBLK, W), jnp.float32),
            pltpu.VMEM((RBLK, W), jnp.float32),
            pltpu.VMEM((RBLK, W), jnp.float32),
            pltpu.VMEM((RBLK, W), jnp.float32),
            pltpu.SemaphoreType.DMA,
            pltpu.SemaphoreType.DMA,
            pltpu.SemaphoreType.DMA,
            pltpu.SemaphoreType.DMA,
        ],
        compiler_params=pltpu.CompilerParams(
            use_tc_tiling_on_sc=True, needs_layout_passes=False
        ),
    )
    def sc_kernel(x_hbm, out_hbm, in0, in1, out0, out1, si0, si1, so0, so1):
        _sc_body(cs, x_hbm, out_hbm, in0, in1, out0, out1, si0, si1, so0, so1)

    return sc_kernel


TCB = 8


def _tc_block(x_ref, o_ref):
    v = x_ref[...]
    neg = -jnp.inf

    parts = []
    carry = None
    for g in range(W // 128 - 1, -1, -1):
        s = v[:, :, 128 * g:128 * (g + 1)]
        for d in (1, 2, 4, 8, 16, 32, 64):
            pad = jnp.full((TCB, H, d), neg, jnp.float32)
            s = jnp.maximum(s, jnp.concatenate([s[:, :, d:], pad], axis=2))
        if carry is not None:
            s = jnp.maximum(s, carry)
        carry = s[:, :, 0:1]
        parts.append(s)
    parts.reverse()
    left = jnp.concatenate(parts, axis=2)

    carry = None
    for k in range(H // 8 - 1, -1, -1):
        s = v[:, 8 * k:8 * k + 8, :]
        for d in (1, 2, 4):
            pad = jnp.full((TCB, d, W), neg, jnp.float32)
            s = jnp.maximum(s, jnp.concatenate([s[:, d:, :], pad], axis=1))
        if carry is not None:
            s = jnp.maximum(s, carry)
        carry = s[:, 0:1, :]
        o_ref[:, 8 * k:8 * k + 8, :] = s + left[:, 8 * k:8 * k + 8, :]


def _tc_kernel(x, c0):
    ct = x.shape[0] - c0
    return pl.pallas_call(
        _tc_block,
        out_shape=jax.ShapeDtypeStruct(x.shape, x.dtype),
        grid=(ct // TCB,),
        in_specs=[pl.BlockSpec((TCB, H, W), lambda i: (i + c0 // TCB, 0, 0))],
        out_specs=pl.BlockSpec((TCB, H, W), lambda i: (i + c0 // TCB, 0, 0)),
    )(x)


CS = 64


def kernel(x):
    if CS == 0:
        return _tc_kernel(x, 0)
    sc = _make_sc_kernel(CS)(x)
    if CS == C:
        return sc
    tc = _tc_kernel(x, CS)
    return lax.dynamic_update_slice(tc, sc, (0, 0, 0))
